# Initial kernel scaffold; baseline (speedup 1.0000x reference)
import jax
import jax.numpy as jnp
from jax.experimental import pallas as pl


def kernel(node_mask_logits, edge_mask_logits, u_node, u_edge, edge_indices, train, return_probs):
    raise NotImplementedError("write your pallas kernel here")


# speedup vs baseline: 3.8435x; 3.8435x over previous
import functools

import jax
import jax.numpy as jnp
from jax import lax
from jax.experimental import pallas as pl
from jax.experimental.pallas import tpu as pltpu
from jax.experimental.pallas import tpu_sc as plsc

P = 4096
E = 327680
EROWS = E // 128
PP = P * P
NS = 16
TE = E // NS
ZCHUNK = 65536
ZPER = PP // NS
NZ = ZPER // ZCHUNK


def _tc_elementwise(nlog_ref, un_ref, elog_ref, ue_ref, i_ref, j_ref, t_ref,
                    node_ref, vals_ref, f1_ref, f2_ref):
    is_train = t_ref[0, 0] != 0

    def gumbel_sigmoid(logits, u):
        g = -jnp.log(-jnp.log(u + 1e-10) + 1e-10)
        return jax.nn.sigmoid(logits + g)

    nlog = nlog_ref[...]
    node_soft = gumbel_sigmoid(nlog, un_ref[...])
    node_hard = (jax.nn.sigmoid(nlog) > 0.5).astype(jnp.float32)
    node_ref[...] = jnp.where(is_train, node_soft, node_hard)

    elog = elog_ref[...]
    edge_soft = gumbel_sigmoid(elog, ue_ref[...])
    edge_hard = (jax.nn.sigmoid(elog) > 0.5).astype(jnp.float32)
    vals_ref[...] = jnp.where(is_train, edge_soft, edge_hard)

    i = i_ref[...]
    j = j_ref[...]
    f1_ref[...] = i * P + j
    f2_ref[...] = j * P + i


def _sc_scatter_body(f1_hbm, f2_hbm, vals_hbm, out_hbm, zbuf, ibuf, vbuf,
                     sem0, sem1):
    sid = lax.axis_index("s")

    def zfill(k, c):
        zbuf[pl.ds(k * 16, 16)] = jnp.zeros((16,), jnp.float32)
        return c

    lax.fori_loop(0, ZCHUNK // 16, zfill, 0)
    zbase = sid * ZPER

    def zcopy(c, carry):
        pltpu.async_copy(zbuf, out_hbm.at[pl.ds(zbase + c * ZCHUNK, ZCHUNK)],
                         sem0)
        return carry

    lax.fori_loop(0, NZ, zcopy, 0)

    def zdrain(c, carry):
        pltpu.make_async_copy(
            zbuf, out_hbm.at[pl.ds(zbase + c * ZCHUNK, ZCHUNK)], sem0).wait()
        return carry

    lax.fori_loop(0, NZ, zdrain, 0)
    plsc.subcore_barrier()

    ebase = sid * TE
    pltpu.sync_copy(f1_hbm.at[pl.ds(ebase, TE)], ibuf)
    pltpu.sync_copy(vals_hbm.at[pl.ds(ebase, TE)], vbuf)
    pltpu.async_copy(vbuf, out_hbm.at[ibuf], sem1).wait()
    plsc.subcore_barrier()

    pltpu.sync_copy(f2_hbm.at[pl.ds(ebase, TE)], ibuf)
    pltpu.async_copy(vbuf, out_hbm.at[ibuf], sem1).wait()


_sc_scatter = functools.partial(
    pl.kernel,
    out_type=jax.ShapeDtypeStruct((PP,), jnp.float32),
    mesh=plsc.VectorSubcoreMesh(core_axis_name="c", subcore_axis_name="s",
                                num_cores=1),
    scratch_types=[
        pltpu.VMEM((ZCHUNK,), jnp.float32),
        pltpu.VMEM((TE,), jnp.int32),
        pltpu.VMEM((TE,), jnp.float32),
        pltpu.SemaphoreType.DMA,
        pltpu.SemaphoreType.DMA,
    ],
)(_sc_scatter_body)


def kernel(node_mask_logits, edge_mask_logits, u_node, u_edge, edge_indices,
           train, return_probs):
    del return_probs
    nlog = node_mask_logits.reshape(P // 128, 128)
    un = u_node.reshape(P // 128, 128)
    elog = edge_mask_logits.reshape(EROWS, 128)
    ue = u_edge.reshape(EROWS, 128)
    ivec = edge_indices[:, 0].reshape(EROWS, 128)
    jvec = edge_indices[:, 1].reshape(EROWS, 128)
    t = jnp.asarray(train, jnp.int32).reshape(1, 1)

    node2d, vals, f1, f2 = pl.pallas_call(
        _tc_elementwise,
        out_shape=(
            jax.ShapeDtypeStruct((P // 128, 128), jnp.float32),
            jax.ShapeDtypeStruct((EROWS, 128), jnp.float32),
            jax.ShapeDtypeStruct((EROWS, 128), jnp.int32),
            jax.ShapeDtypeStruct((EROWS, 128), jnp.int32),
        ),
        in_specs=[
            pl.BlockSpec(memory_space=pltpu.VMEM),
            pl.BlockSpec(memory_space=pltpu.VMEM),
            pl.BlockSpec(memory_space=pltpu.VMEM),
            pl.BlockSpec(memory_space=pltpu.VMEM),
            pl.BlockSpec(memory_space=pltpu.VMEM),
            pl.BlockSpec(memory_space=pltpu.VMEM),
            pl.BlockSpec(memory_space=pltpu.SMEM),
        ],
    )(nlog, un, elog, ue, ivec, jvec, t)

    out_flat = _sc_scatter(f1.reshape(E), f2.reshape(E), vals.reshape(E))
    return node2d.reshape(P), out_flat.reshape(P, P)
